# baseline (speedup 1.0000x reference)
import functools

import jax
import jax.numpy as jnp
from jax import lax
from jax.experimental import pallas as pl
from jax.experimental.pallas import tpu as pltpu
from jax.experimental.pallas import tpu_sc as plsc

E = 8
D = 1024
F = 2816
T = 2048
P = 2 * T

TM = 256
G = P // TM + (E - 1)
GT = G * TM
FT = 1408
NF = F // FT
FJ = 352

NC = 2
NS = 16
NW = NC * NS
PPW = P // NW
DCH = 64
TPW = T // NW
CH = 32

def _sc_mesh():
    return plsc.VectorSubcoreMesh(core_axis_name="c", subcore_axis_name="s")


def _router_kernel(x_ref, wg_ref, logits_ref, meta_ref):
    x = x_ref[...]
    wg = wg_ref[...] * 0.5
    logits = lax.dot_general(
        x, wg, (((1,), (0,)), ((), ())),
        preferred_element_type=jnp.float32,
        precision=lax.Precision.DEFAULT)
    logits_ref[...] = logits
    m = jnp.max(logits, axis=1, keepdims=True)
    ex = jnp.exp(logits - m)
    probs = ex / jnp.sum(ex, axis=1, keepdims=True)
    idx = lax.broadcasted_iota(jnp.int32, probs.shape, 1)
    m1 = jnp.max(probs, axis=1, keepdims=True)
    i1 = jnp.min(jnp.where(probs == m1, idx, E), axis=1, keepdims=True)
    pm = jnp.where(idx == i1, -jnp.inf, probs)
    m2 = jnp.max(pm, axis=1, keepdims=True)
    i2 = jnp.min(jnp.where(pm == m2, idx, E), axis=1, keepdims=True)
    s = m1 + m2
    lidx = lax.broadcasted_iota(jnp.int32, meta_ref.shape, 1)
    meta_ref[...] = jnp.where(
        lidx == 0, i1.astype(jnp.float32),
        jnp.where(lidx == 1, i2.astype(jnp.float32),
                  jnp.where(lidx == 2, m1 / s, m2 / s)))


def _sc_dispatch(x_hbm, pos_hbm, xpad_hbm, idx_v, rows_v, sem):
    wid = lax.axis_index("s") * NC + lax.axis_index("c")
    base = wid * PPW

    @pl.loop(0, PPW // DCH)
    def _(c):
        b = base + c * DCH
        pltpu.sync_copy(pos_hbm.at[pl.ds(b, DCH)], idx_v)
        pltpu.sync_copy(x_hbm.at[pl.ds(lax.rem(b, T), DCH)], rows_v)
        pltpu.async_copy(rows_v, xpad_hbm.at[idx_v], sem).wait()


def _sc_combine(opad_hbm, pos1_hbm, pos2_hbm, g1_hbm, g2_hbm,
                idx1_v, idx2_v, r1_v, r2_v, sem1, sem2):
    wid = lax.axis_index("s") * NC + lax.axis_index("c")
    base = wid * TPW

    @pl.loop(0, TPW // CH)
    def _(c):
        b = base + c * CH
        pltpu.sync_copy(pos1_hbm.at[pl.ds(b, CH)], idx1_v)
        pltpu.sync_copy(pos2_hbm.at[pl.ds(b, CH)], idx2_v)
        cp1 = pltpu.async_copy(opad_hbm.at[idx1_v], r1_v, sem1)
        cp2 = pltpu.async_copy(opad_hbm.at[idx2_v], r2_v, sem2)
        cp1.wait()
        cp2.wait()
        pltpu.sync_copy(r1_v, g1_hbm.at[pl.ds(b, CH)])
        pltpu.sync_copy(r2_v, g2_hbm.at[pl.ds(b, CH)])


def _moe_mm_kernel(te_ref, sched_ref, x_ref, w1_hbm, w2_hbm, w3_hbm, out_ref,
                   w1b, w2b, w3b, sems, acc_ref):
    f = pl.program_id(0)
    i = pl.program_id(1)
    k = f * G + i
    sl = pl.ds(i * TM, TM)
    rs = sched_ref[0, k]
    b = sched_ref[1, k]
    fetch = sched_ref[2, k]
    fe = sched_ref[3, k]
    ff = sched_ref[4, k]
    fb = sched_ref[5, k]

    def _copies(e, fc, slot):
        fsl = pl.ds(fc * FT, FT)
        return (
            pltpu.make_async_copy(w1_hbm.at[e, fsl, :], w1b.at[slot], sems.at[slot]),
            pltpu.make_async_copy(w2_hbm.at[e, fsl, :], w2b.at[slot], sems.at[slot]),
            pltpu.make_async_copy(w3_hbm.at[e, :, fsl], w3b.at[slot], sems.at[slot]),
        )

    @pl.when(k == 0)
    def _():
        for cp in _copies(te_ref[0], 0, 0):
            cp.start()

    @pl.when(fetch == 1)
    def _():
        for cp in _copies(fe, ff, fb):
            cp.start()

    @pl.when(rs == 1)
    def _():
        for cp in _copies(te_ref[i], f, b):
            cp.wait()

    @pl.when(i < te_ref[G])
    def _():
        x = x_ref[...]
        o = jnp.zeros((TM, D), jnp.float32)
        for j in range(FT // FJ):
            jsl = pl.ds(j * FJ, FJ)
            w1 = w1b[b, jsl, :]
            w2 = w2b[b, jsl, :]
            w3 = w3b[b, :, jsl]
            h1 = lax.dot_general(
                x, w1, (((1,), (1,)), ((), ())),
                preferred_element_type=jnp.float32,
                precision=lax.Precision.DEFAULT)
            h2 = lax.dot_general(
                x, w2, (((1,), (1,)), ((), ())),
                preferred_element_type=jnp.float32,
                precision=lax.Precision.DEFAULT)
            h = (h1 * jax.nn.sigmoid(h1)) * h2
            o = o + lax.dot_general(
                h, w3, (((1,), (1,)), ((), ())),
                preferred_element_type=jnp.float32,
                precision=lax.Precision.DEFAULT)

        @pl.when(f == 0)
        def _():
            acc_ref[sl, :] = o.astype(jnp.bfloat16)

        @pl.when(f == NF - 1)
        def _():
            out_ref[...] = acc_ref[sl, :].astype(jnp.float32) + o


def _wadd_kernel(g1_ref, g2_ref, meta_ref, out_ref):
    lidx = lax.broadcasted_iota(jnp.int32, meta_ref.shape, 1)
    meta = meta_ref[...]
    w1 = jnp.sum(jnp.where(lidx == 2, meta, 0.0), axis=1, keepdims=True)
    w2 = jnp.sum(jnp.where(lidx == 3, meta, 0.0), axis=1, keepdims=True)
    out_ref[...] = g1_ref[...] * w1 + g2_ref[...] * w2


def _dispatch_call(x, pos_pairs):
    run = functools.partial(
        pl.kernel,
        out_type=jax.ShapeDtypeStruct((GT, D), jnp.float32),
        mesh=_sc_mesh(),
        scratch_types=[
            pltpu.VMEM((DCH,), jnp.int32),
            pltpu.VMEM((DCH, D), jnp.float32),
            pltpu.SemaphoreType.DMA,
        ],
    )(_sc_dispatch)
    return run(x, pos_pairs)


def _combine_call(out_pad, pos1, pos2):
    run = functools.partial(
        pl.kernel,
        out_type=(
            jax.ShapeDtypeStruct((T, D), jnp.float32),
            jax.ShapeDtypeStruct((T, D), jnp.float32),
        ),
        mesh=_sc_mesh(),
        scratch_types=[
            pltpu.VMEM((CH,), jnp.int32),
            pltpu.VMEM((CH,), jnp.int32),
            pltpu.VMEM((CH, D), jnp.float32),
            pltpu.VMEM((CH, D), jnp.float32),
            pltpu.SemaphoreType.DMA,
            pltpu.SemaphoreType.DMA,
        ],
    )(_sc_combine)
    return run(out_pad, pos1, pos2)


def _moe_mm_call(te, sched, x_pad, fc1_1, fc1_2, fc2):
    return pl.pallas_call(
        _moe_mm_kernel,
        grid_spec=pltpu.PrefetchScalarGridSpec(
            num_scalar_prefetch=2,
            grid=(NF, G),
            in_specs=[
                pl.BlockSpec((TM, D), lambda f, i, te, sc: (i, 0)),
                pl.BlockSpec(memory_space=pl.ANY),
                pl.BlockSpec(memory_space=pl.ANY),
                pl.BlockSpec(memory_space=pl.ANY),
            ],
            out_specs=pl.BlockSpec(
                (TM, D), lambda f, i, te, sc: (jnp.where(f == NF - 1, i, 0), 0)),
            scratch_shapes=[
                pltpu.VMEM((2, FT, D), jnp.float32),
                pltpu.VMEM((2, FT, D), jnp.float32),
                pltpu.VMEM((2, D, FT), jnp.float32),
                pltpu.SemaphoreType.DMA((2,)),
                pltpu.VMEM((GT, D), jnp.bfloat16),
            ],
        ),
        out_shape=jax.ShapeDtypeStruct((GT, D), jnp.float32),
    )(te, sched, x_pad, fc1_1, fc1_2, fc2)


def kernel(hidden_states, wg, fc1_1, fc1_2, fc2):
    B, S, _ = hidden_states.shape
    x = hidden_states.reshape(T, D)

    logits, meta = pl.pallas_call(
        _router_kernel,
        out_shape=(
            jax.ShapeDtypeStruct((T, E), jnp.float32),
            jax.ShapeDtypeStruct((T, 4), jnp.float32),
        ),
    )(x, wg)

    def _psum0(a):
        n = a.shape[0]
        s = 1
        while s < n:
            a = a + jnp.pad(a, ((s, 0), (0, 0)))[:n]
            s *= 2
        return a

    i1 = meta[:, 0].astype(jnp.int32)
    i2 = meta[:, 1].astype(jnp.int32)
    eidx = jnp.arange(E, dtype=jnp.int32)[None, :]
    oh1 = (i1[:, None] == eidx).astype(jnp.int32)
    oh2 = (i2[:, None] == eidx).astype(jnp.int32)
    c1 = _psum0(oh1)
    c2 = _psum0(oh2)
    s1 = c1[-1]
    counts = s1 + c2[-1]
    nt = (counts + TM - 1) // TM
    tb = _psum0(nt[:, None])[:, 0] - nt
    base_e = tb * TM
    rank1 = jnp.sum((c1 - oh1) * oh1, axis=1)
    rank2 = jnp.sum((s1[None, :] + c2 - oh2) * oh2, axis=1)
    pos1 = jnp.sum(oh1 * base_e[None, :], axis=1) + rank1
    pos2 = jnp.sum(oh2 * base_e[None, :], axis=1) + rank2
    pos_pairs = jnp.concatenate([pos1, pos2]).astype(jnp.int32)
    gidx = jnp.arange(G, dtype=jnp.int32)[:, None]
    te = jnp.minimum(
        jnp.sum(((tb + nt)[None, :] <= gidx).astype(jnp.int32), axis=1), E - 1
    ).astype(jnp.int32)
    ntot = jnp.sum(nt, keepdims=True).astype(jnp.int32)

    ik = jnp.tile(jnp.arange(G, dtype=jnp.int32), NF)
    fk = jnp.repeat(jnp.arange(NF, dtype=jnp.int32), G)
    exk = jnp.tile(te, NF)
    newrun = jnp.concatenate([
        jnp.ones((1,), jnp.int32),
        ((ik[1:] == 0) | (exk[1:] != exk[:-1])).astype(jnp.int32)])
    rid = _psum0(newrun[:, None])[:, 0] - 1
    buf = rid % 2
    nxt = jnp.sum((te[None, :] <= te[:, None]).astype(jnp.int32), axis=1)
    nxtk = jnp.tile(nxt, NF)
    last_run = (fk == NF - 1) & (nxtk >= G)
    fe = jnp.where(nxtk < G, jnp.tile(te[jnp.minimum(nxtk, G - 1)], 1), te[0])
    ff = jnp.where(nxtk < G, fk, fk + 1)
    fetch = (newrun == 1) & (~last_run)
    sched = jnp.stack([
        newrun,
        buf,
        fetch.astype(jnp.int32),
        fe,
        jnp.minimum(ff, NF - 1),
        1 - buf,
    ]).astype(jnp.int32)

    te = jnp.concatenate([te, ntot])

    x_pad = _dispatch_call(x, pos_pairs)
    out_pad = _moe_mm_call(te, sched, x_pad, fc1_1, fc1_2, fc2)
    g1, g2 = _combine_call(out_pad, pos1.astype(jnp.int32), pos2.astype(jnp.int32))

    final = pl.pallas_call(
        _wadd_kernel,
        grid=(4,),
        in_specs=[
            pl.BlockSpec((T // 4, D), lambda i: (i, 0)),
            pl.BlockSpec((T // 4, D), lambda i: (i, 0)),
            pl.BlockSpec((T // 4, 4), lambda i: (i, 0)),
        ],
        out_specs=pl.BlockSpec((T // 4, D), lambda i: (i, 0)),
        out_shape=jax.ShapeDtypeStruct((T, D), jnp.float32),
    )(g1, g2, meta)

    return final.reshape(B, S, D), logits


# speedup vs baseline: 1.1312x; 1.1312x over previous
import functools

import jax
import jax.numpy as jnp
from jax import lax
from jax.experimental import pallas as pl
from jax.experimental.pallas import tpu as pltpu
from jax.experimental.pallas import tpu_sc as plsc

E = 8
D = 1024
F = 2816
T = 2048
P = 2 * T

TM = 256
G = P // TM + (E - 1)
GT = G * TM
FT = 1408
NF = F // FT

NC = 2
NS = 16
NW = NC * NS
PPW = P // NW
DCH = 64
TPW = T // NW
CH = 32

def _sc_mesh():
    return plsc.VectorSubcoreMesh(core_axis_name="c", subcore_axis_name="s")


def _router_kernel(x_ref, wg_ref, logits_ref, meta_ref):
    x = x_ref[...]
    wg = wg_ref[...] * 0.5
    logits = lax.dot_general(
        x, wg, (((1,), (0,)), ((), ())),
        preferred_element_type=jnp.float32,
        precision=lax.Precision.DEFAULT)
    logits_ref[...] = logits
    m = jnp.max(logits, axis=1, keepdims=True)
    ex = jnp.exp(logits - m)
    probs = ex / jnp.sum(ex, axis=1, keepdims=True)
    idx = lax.broadcasted_iota(jnp.int32, probs.shape, 1)
    m1 = jnp.max(probs, axis=1, keepdims=True)
    i1 = jnp.min(jnp.where(probs == m1, idx, E), axis=1, keepdims=True)
    pm = jnp.where(idx == i1, -jnp.inf, probs)
    m2 = jnp.max(pm, axis=1, keepdims=True)
    i2 = jnp.min(jnp.where(pm == m2, idx, E), axis=1, keepdims=True)
    s = m1 + m2
    lidx = lax.broadcasted_iota(jnp.int32, meta_ref.shape, 1)
    meta_ref[...] = jnp.where(
        lidx == 0, i1.astype(jnp.float32),
        jnp.where(lidx == 1, i2.astype(jnp.float32),
                  jnp.where(lidx == 2, m1 / s, m2 / s)))


def _sc_dispatch(x_hbm, pos_hbm, xpad_hbm, idx_v, rows_v, sem):
    wid = lax.axis_index("s") * NC + lax.axis_index("c")
    base = wid * PPW

    @pl.loop(0, PPW // DCH)
    def _(c):
        b = base + c * DCH
        pltpu.sync_copy(pos_hbm.at[pl.ds(b, DCH)], idx_v)
        pltpu.sync_copy(x_hbm.at[pl.ds(lax.rem(b, T), DCH)], rows_v)
        pltpu.async_copy(rows_v, xpad_hbm.at[idx_v], sem).wait()


def _sc_combine(opad_hbm, pos1_hbm, pos2_hbm, g1_hbm, g2_hbm,
                idx1_v, idx2_v, r1_v, r2_v, sem1, sem2):
    wid = lax.axis_index("s") * NC + lax.axis_index("c")
    base = wid * TPW

    @pl.loop(0, TPW // CH)
    def _(c):
        b = base + c * CH
        pltpu.sync_copy(pos1_hbm.at[pl.ds(b, CH)], idx1_v)
        pltpu.sync_copy(pos2_hbm.at[pl.ds(b, CH)], idx2_v)
        cp1 = pltpu.async_copy(opad_hbm.at[idx1_v], r1_v, sem1)
        cp2 = pltpu.async_copy(opad_hbm.at[idx2_v], r2_v, sem2)
        cp1.wait()
        cp2.wait()
        pltpu.sync_copy(r1_v, g1_hbm.at[pl.ds(b, CH)])
        pltpu.sync_copy(r2_v, g2_hbm.at[pl.ds(b, CH)])


def _moe_mm_kernel(te_ref, sched_ref, x_ref, w1_hbm, w2_hbm, w3_hbm, out_ref,
                   w1b, w2b, w3b, sems, acc_ref):
    f = pl.program_id(0)
    i = pl.program_id(1)
    k = f * G + i
    sl = pl.ds(i * TM, TM)
    rs = sched_ref[0, k]
    b = sched_ref[1, k]
    fetch = sched_ref[2, k]
    fe = sched_ref[3, k]
    ff = sched_ref[4, k]
    fb = sched_ref[5, k]

    def _copies(e, fc, slot):
        fsl = pl.ds(fc * FT, FT)
        return (
            pltpu.make_async_copy(w1_hbm.at[e, fsl, :], w1b.at[slot], sems.at[slot]),
            pltpu.make_async_copy(w2_hbm.at[e, fsl, :], w2b.at[slot], sems.at[slot]),
            pltpu.make_async_copy(w3_hbm.at[e, :, fsl], w3b.at[slot], sems.at[slot]),
        )

    @pl.when(k == 0)
    def _():
        for cp in _copies(te_ref[0], 0, 0):
            cp.start()

    @pl.when(fetch == 1)
    def _():
        for cp in _copies(fe, ff, fb):
            cp.start()

    @pl.when(rs == 1)
    def _():
        for cp in _copies(te_ref[i], f, b):
            cp.wait()

    @pl.when(i < te_ref[G])
    def _():
        x = x_ref[...]
        w1 = w1b[b]
        w2 = w2b[b]
        w3 = w3b[b]
        h1 = lax.dot_general(
            x, w1, (((1,), (1,)), ((), ())),
            preferred_element_type=jnp.float32, precision=lax.Precision.DEFAULT)
        h2 = lax.dot_general(
            x, w2, (((1,), (1,)), ((), ())),
            preferred_element_type=jnp.float32, precision=lax.Precision.DEFAULT)
        h = (h1 * jax.nn.sigmoid(h1)) * h2
        o = lax.dot_general(
            h, w3, (((1,), (1,)), ((), ())),
            preferred_element_type=jnp.float32, precision=lax.Precision.DEFAULT)

        @pl.when(f == 0)
        def _():
            acc_ref[sl, :] = o.astype(jnp.bfloat16)

        @pl.when(f == NF - 1)
        def _():
            out_ref[...] = acc_ref[sl, :].astype(jnp.float32) + o


def _wadd_kernel(g1_ref, g2_ref, meta_ref, out_ref):
    lidx = lax.broadcasted_iota(jnp.int32, meta_ref.shape, 1)
    meta = meta_ref[...]
    w1 = jnp.sum(jnp.where(lidx == 2, meta, 0.0), axis=1, keepdims=True)
    w2 = jnp.sum(jnp.where(lidx == 3, meta, 0.0), axis=1, keepdims=True)
    out_ref[...] = g1_ref[...] * w1 + g2_ref[...] * w2


def _dispatch_call(x, pos_pairs):
    run = functools.partial(
        pl.kernel,
        out_type=jax.ShapeDtypeStruct((GT, D), jnp.float32),
        mesh=_sc_mesh(),
        scratch_types=[
            pltpu.VMEM((DCH,), jnp.int32),
            pltpu.VMEM((DCH, D), jnp.float32),
            pltpu.SemaphoreType.DMA,
        ],
    )(_sc_dispatch)
    return run(x, pos_pairs)


def _combine_call(out_pad, pos1, pos2):
    run = functools.partial(
        pl.kernel,
        out_type=(
            jax.ShapeDtypeStruct((T, D), jnp.float32),
            jax.ShapeDtypeStruct((T, D), jnp.float32),
        ),
        mesh=_sc_mesh(),
        scratch_types=[
            pltpu.VMEM((CH,), jnp.int32),
            pltpu.VMEM((CH,), jnp.int32),
            pltpu.VMEM((CH, D), jnp.float32),
            pltpu.VMEM((CH, D), jnp.float32),
            pltpu.SemaphoreType.DMA,
            pltpu.SemaphoreType.DMA,
        ],
    )(_sc_combine)
    return run(out_pad, pos1, pos2)


def _moe_mm_call(te, sched, x_pad, fc1_1, fc1_2, fc2):
    return pl.pallas_call(
        _moe_mm_kernel,
        grid_spec=pltpu.PrefetchScalarGridSpec(
            num_scalar_prefetch=2,
            grid=(NF, G),
            in_specs=[
                pl.BlockSpec((TM, D), lambda f, i, te, sc: (i, 0)),
                pl.BlockSpec(memory_space=pl.ANY),
                pl.BlockSpec(memory_space=pl.ANY),
                pl.BlockSpec(memory_space=pl.ANY),
            ],
            out_specs=pl.BlockSpec(
                (TM, D), lambda f, i, te, sc: (jnp.where(f == NF - 1, i, 0), 0)),
            scratch_shapes=[
                pltpu.VMEM((2, FT, D), jnp.float32),
                pltpu.VMEM((2, FT, D), jnp.float32),
                pltpu.VMEM((2, D, FT), jnp.float32),
                pltpu.SemaphoreType.DMA((2,)),
                pltpu.VMEM((GT, D), jnp.bfloat16),
            ],
        ),
        out_shape=jax.ShapeDtypeStruct((GT, D), jnp.float32),
    )(te, sched, x_pad, fc1_1, fc1_2, fc2)


def kernel(hidden_states, wg, fc1_1, fc1_2, fc2):
    B, S, _ = hidden_states.shape
    x = hidden_states.reshape(T, D)

    logits, meta = pl.pallas_call(
        _router_kernel,
        out_shape=(
            jax.ShapeDtypeStruct((T, E), jnp.float32),
            jax.ShapeDtypeStruct((T, 4), jnp.float32),
        ),
    )(x, wg)

    def _psum0(a):
        n = a.shape[0]
        s = 1
        while s < n:
            a = a + jnp.pad(a, ((s, 0), (0, 0)))[:n]
            s *= 2
        return a

    i1 = meta[:, 0].astype(jnp.int32)
    i2 = meta[:, 1].astype(jnp.int32)
    eidx = jnp.arange(E, dtype=jnp.int32)[None, :]
    oh1 = (i1[:, None] == eidx).astype(jnp.int32)
    oh2 = (i2[:, None] == eidx).astype(jnp.int32)
    c1 = _psum0(oh1)
    c2 = _psum0(oh2)
    s1 = c1[-1]
    counts = s1 + c2[-1]
    nt = (counts + TM - 1) // TM
    tb = _psum0(nt[:, None])[:, 0] - nt
    base_e = tb * TM
    rank1 = jnp.sum((c1 - oh1) * oh1, axis=1)
    rank2 = jnp.sum((s1[None, :] + c2 - oh2) * oh2, axis=1)
    pos1 = jnp.sum(oh1 * base_e[None, :], axis=1) + rank1
    pos2 = jnp.sum(oh2 * base_e[None, :], axis=1) + rank2
    pos_pairs = jnp.concatenate([pos1, pos2]).astype(jnp.int32)
    gidx = jnp.arange(G, dtype=jnp.int32)[:, None]
    te = jnp.minimum(
        jnp.sum(((tb + nt)[None, :] <= gidx).astype(jnp.int32), axis=1), E - 1
    ).astype(jnp.int32)
    ntot = jnp.sum(nt, keepdims=True).astype(jnp.int32)

    ik = jnp.tile(jnp.arange(G, dtype=jnp.int32), NF)
    fk = jnp.repeat(jnp.arange(NF, dtype=jnp.int32), G)
    exk = jnp.tile(te, NF)
    newrun = jnp.concatenate([
        jnp.ones((1,), jnp.int32),
        ((ik[1:] == 0) | (exk[1:] != exk[:-1])).astype(jnp.int32)])
    rid = _psum0(newrun[:, None])[:, 0] - 1
    buf = rid % 2
    nxt = jnp.sum((te[None, :] <= te[:, None]).astype(jnp.int32), axis=1)
    nxtk = jnp.tile(nxt, NF)
    last_run = (fk == NF - 1) & (nxtk >= G)
    fe = jnp.where(nxtk < G, jnp.tile(te[jnp.minimum(nxtk, G - 1)], 1), te[0])
    ff = jnp.where(nxtk < G, fk, fk + 1)
    fetch = (newrun == 1) & (~last_run)
    sched = jnp.stack([
        newrun,
        buf,
        fetch.astype(jnp.int32),
        fe,
        jnp.minimum(ff, NF - 1),
        1 - buf,
    ]).astype(jnp.int32)

    te = jnp.concatenate([te, ntot])

    x_pad = _dispatch_call(x, pos_pairs)
    out_pad = _moe_mm_call(te, sched, x_pad, fc1_1, fc1_2, fc2)
    g1, g2 = _combine_call(out_pad, pos1.astype(jnp.int32), pos2.astype(jnp.int32))

    final = pl.pallas_call(
        _wadd_kernel,
        grid=(4,),
        in_specs=[
            pl.BlockSpec((T // 4, D), lambda i: (i, 0)),
            pl.BlockSpec((T // 4, D), lambda i: (i, 0)),
            pl.BlockSpec((T // 4, 4), lambda i: (i, 0)),
        ],
        out_specs=pl.BlockSpec((T // 4, D), lambda i: (i, 0)),
        out_shape=jax.ShapeDtypeStruct((T, D), jnp.float32),
    )(g1, g2, meta)

    return final.reshape(B, S, D), logits
